# baseline (speedup 1.0000x reference)
import numpy as np
import jax
import jax.numpy as jnp
from jax.experimental import pallas as pl
from jax.experimental.pallas import tpu as pltpu

_CLUB = np.array([[0, 1, 1, 1],
                  [1, 0, 0, 0],
                  [1, 0, 0, 1],
                  [1, 0, 1, 0]], dtype=np.float32)
_DEG = (_CLUB + np.eye(4, dtype=np.float32)).sum(axis=0)
_DINV = 1.0 / np.sqrt(_DEG)
_A = ((_CLUB + np.eye(4, dtype=np.float32)) * _DINV[None, :] * _DINV[:, None]).astype(np.float32)

_TILE = 25000


def _body(h_ref, W_ref, b_ref, o_ref):
    h_blk = h_ref[...]
    rows = [h_blk[s:s + 1, :] for s in range(4)]
    mixed = jnp.concatenate(
        [sum(float(_A[d, s]) * rows[s] for s in range(4) if _A[d, s] != 0.0)
         for d in range(4)],
        axis=0,
    )
    h_fixed = jnp.concatenate([mixed, h_blk[4:, :]], axis=0)
    h_use = jnp.where(pl.program_id(0) == 0, h_fixed, h_blk)
    x = jnp.dot(h_use, W_ref[...], preferred_element_type=jnp.float32)
    o_ref[...] = jnp.tanh(x + b_ref[...])


def kernel(t, h, W, b):
    del t
    N, D = h.shape
    grid = (N // _TILE,)
    out = pl.pallas_call(
        _body,
        grid=grid,
        in_specs=[
            pl.BlockSpec((_TILE, D), lambda i: (i, 0)),
            pl.BlockSpec((D, D), lambda i: (0, 0)),
            pl.BlockSpec((1, D), lambda i: (0, 0)),
        ],
        out_specs=pl.BlockSpec((_TILE, D), lambda i: (i, 0)),
        out_shape=jax.ShapeDtypeStruct((N, D), jnp.float32),
        compiler_params=pltpu.CompilerParams(
            dimension_semantics=("arbitrary",),
        ),
    )(h, W, b.reshape(1, D))
    return out


# speedup vs baseline: 1.0221x; 1.0221x over previous
import numpy as np
import jax
import jax.numpy as jnp
from jax.experimental import pallas as pl
from jax.experimental.pallas import tpu as pltpu

_CLUB = np.array([[0, 1, 1, 1],
                  [1, 0, 0, 0],
                  [1, 0, 0, 1],
                  [1, 0, 1, 0]], dtype=np.float32)
_DEG = (_CLUB + np.eye(4, dtype=np.float32)).sum(axis=0)
_DINV = 1.0 / np.sqrt(_DEG)
_A = ((_CLUB + np.eye(4, dtype=np.float32)) * _DINV[None, :] * _DINV[:, None]).astype(np.float32)

_TILE = 10000


def _body(h_ref, W_ref, b_ref, o_ref):
    h_blk = h_ref[...]
    rows = [h_blk[s:s + 1, :] for s in range(4)]
    mixed = jnp.concatenate(
        [sum(float(_A[d, s]) * rows[s] for s in range(4) if _A[d, s] != 0.0)
         for d in range(4)],
        axis=0,
    )
    h_fixed = jnp.concatenate([mixed, h_blk[4:, :]], axis=0)
    h_use = jnp.where(pl.program_id(0) == 0, h_fixed, h_blk)
    x = jnp.dot(h_use, W_ref[...], preferred_element_type=jnp.float32)
    o_ref[...] = jnp.tanh(x + b_ref[...])


def kernel(t, h, W, b):
    del t
    N, D = h.shape
    grid = (N // _TILE,)
    out = pl.pallas_call(
        _body,
        grid=grid,
        in_specs=[
            pl.BlockSpec((_TILE, D), lambda i: (i, 0)),
            pl.BlockSpec((D, D), lambda i: (0, 0)),
            pl.BlockSpec((1, D), lambda i: (0, 0)),
        ],
        out_specs=pl.BlockSpec((_TILE, D), lambda i: (i, 0)),
        out_shape=jax.ShapeDtypeStruct((N, D), jnp.float32),
        compiler_params=pltpu.CompilerParams(
            dimension_semantics=("parallel",),
        ),
    )(h, W, b.reshape(1, D))
    return out


# speedup vs baseline: 1.0693x; 1.0462x over previous
import numpy as np
import jax
import jax.numpy as jnp
from jax.experimental import pallas as pl
from jax.experimental.pallas import tpu as pltpu

_CLUB = np.array([[0, 1, 1, 1],
                  [1, 0, 0, 0],
                  [1, 0, 0, 1],
                  [1, 0, 1, 0]], dtype=np.float32)
_DEG = (_CLUB + np.eye(4, dtype=np.float32)).sum(axis=0)
_DINV = 1.0 / np.sqrt(_DEG)
_A = ((_CLUB + np.eye(4, dtype=np.float32)) * _DINV[None, :] * _DINV[:, None]).astype(np.float32)

_TILE = 20000


def _body(h_ref, W_ref, b_ref, o_ref):
    h_blk = h_ref[...]
    rows = [h_blk[s:s + 1, :] for s in range(4)]
    mixed = jnp.concatenate(
        [sum(float(_A[d, s]) * rows[s] for s in range(4) if _A[d, s] != 0.0)
         for d in range(4)],
        axis=0,
    )
    h_fixed = jnp.concatenate([mixed, h_blk[4:, :]], axis=0)
    h_use = jnp.where(pl.program_id(0) == 0, h_fixed, h_blk)
    x = jnp.dot(h_use, W_ref[...], preferred_element_type=jnp.float32)
    o_ref[...] = jnp.tanh(x + b_ref[...])


def kernel(t, h, W, b):
    del t
    N, D = h.shape
    grid = (N // _TILE,)
    out = pl.pallas_call(
        _body,
        grid=grid,
        in_specs=[
            pl.BlockSpec((_TILE, D), lambda i: (i, 0)),
            pl.BlockSpec((D, D), lambda i: (0, 0)),
            pl.BlockSpec((1, D), lambda i: (0, 0)),
        ],
        out_specs=pl.BlockSpec((_TILE, D), lambda i: (i, 0)),
        out_shape=jax.ShapeDtypeStruct((N, D), jnp.float32),
        compiler_params=pltpu.CompilerParams(
            dimension_semantics=("parallel",),
        ),
    )(h, W, b.reshape(1, D))
    return out
